# baseline (speedup 1.0000x reference)
import functools

import jax
import jax.numpy as jnp
from jax import lax
from jax.experimental import pallas as pl
from jax.experimental.pallas import tpu as pltpu
from jax.experimental.pallas import tpu_sc as plsc

N = 10000
E = 160000
B = 1024

_info = plsc.get_sparse_core_info()
NC = _info.num_cores
NS = _info.num_subcores
L = _info.num_lanes
NW = NC * NS

W = 8
NSP = B // W
NPASS = NSP // NW

NB = 1024
NBLK = (N + NB - 1) // NB


def _tc_prep_body(x_ref, h_ref, xq_ref, xh_ref):
    pid = pl.program_id(1)
    col = pid * NB + lax.broadcasted_iota(jnp.int32, (1, NB), 1)
    valid = col < N
    xb = jnp.where(valid, x_ref[...], 0.0)
    hb = jnp.where(valid, h_ref[...], 0.0)
    xq_ref[...] = xb.T.reshape(1, NB, W)

    @pl.when(pid == 0)
    def _():
        xh_ref[...] = jnp.zeros_like(xh_ref)

    xh_ref[...] += lax.dot_general(
        xb, hb.reshape(NB, 1), (((1,), (0,)), ((), ())),
        preferred_element_type=jnp.float32)


_tc_prep = pl.pallas_call(
    _tc_prep_body,
    grid=(NSP, NBLK),
    in_specs=[
        pl.BlockSpec((W, NB), lambda s, i: (s, i)),
        pl.BlockSpec((1, NB), lambda s, i: (0, i)),
    ],
    out_specs=[
        pl.BlockSpec((1, NB, W), lambda s, i: (s, i, 0)),
        pl.BlockSpec((W, 1), lambda s, i: (s, 0)),
    ],
    out_shape=[
        jax.ShapeDtypeStruct((NSP, N, W), jnp.float32),
        jax.ShapeDtypeStruct((B, 1), jnp.float32),
    ],
)


K = 2000
NCH = E // K
NGRP = K // L
SD = 2


@functools.partial(
    pl.kernel,
    mesh=plsc.VectorSubcoreMesh(core_axis_name="c", subcore_axis_name="s"),
    compiler_params=pltpu.CompilerParams(needs_layout_passes=False),
    out_type=jax.ShapeDtypeStruct((B, L), jnp.float32),
    scratch_types=[
        pltpu.VMEM((N * W,), jnp.float32),
        pltpu.VMEM((K,), jnp.int32),
        pltpu.VMEM((K,), jnp.int32),
        pltpu.VMEM((K,), jnp.int32),
        pltpu.VMEM((K,), jnp.int32),
        pltpu.VMEM((K,), jnp.float32),
        pltpu.VMEM((K,), jnp.float32),
        pltpu.VMEM((W, L), jnp.float32),
        pltpu.SemaphoreType.DMA,
        pltpu.SemaphoreType.DMA,
        pltpu.SemaphoreType.DMA,
    ],
)
def _sc_edges(xq_hbm, ei_hbm, ej_hbm, j_hbm, out_hbm,
              tab, ei0, ei1, ej0, ej1, jv0, jv1, accbuf, *sems):
    wid = lax.axis_index("s") * NC + lax.axis_index("c")
    ei_s, ej_s, jv_s = (ei0, ei1), (ej0, ej1), (jv0, jv1)

    def fire(ch, slot):
        off = ch * K
        pltpu.async_copy(ei_hbm.at[pl.ds(off, K)], ei_s[slot], sems[slot])
        pltpu.async_copy(ej_hbm.at[pl.ds(off, K)], ej_s[slot], sems[slot])
        pltpu.async_copy(j_hbm.at[pl.ds(off, K)], jv_s[slot], sems[slot])

    def drain(slot):
        pltpu.make_async_copy(ei_hbm.at[pl.ds(0, K)], ei_s[slot],
                              sems[slot]).wait()
        pltpu.make_async_copy(ej_hbm.at[pl.ds(0, K)], ej_s[slot],
                              sems[slot]).wait()
        pltpu.make_async_copy(j_hbm.at[pl.ds(0, K)], jv_s[slot],
                              sems[slot]).wait()

    wcols = [jnp.full((L,), w, jnp.int32) for w in range(W)]

    for p in range(NPASS):
        sp = p * NW + wid
        pltpu.async_copy(xq_hbm.at[sp], tab, sems[2]).wait()
        fire(0, 0)

        def process(ch, accs, slot):
            drain(slot)

            @pl.when(ch + 1 < NCH)
            def _():
                fire(ch + 1, 1 - slot)

            def group_body(g, accs2, _slot=slot):
                ai = lax.shift_left(ei_s[_slot][pl.ds(g * L, L)], 3)
                aj = lax.shift_left(ej_s[_slot][pl.ds(g * L, L)], 3)
                j16 = jv_s[_slot][pl.ds(g * L, L)]
                out = []
                for w in range(W):
                    gi = plsc.load_gather(tab, [ai + wcols[w]])
                    gj = plsc.load_gather(tab, [aj + wcols[w]])
                    out.append(accs2[w] + gi * gj * j16)
                return tuple(out)

            return lax.fori_loop(0, NGRP, group_body, accs)

        def chunk_pair(o, accs, _p=p):
            accs = process(2 * o, accs, 0)
            accs = process(2 * o + 1, accs, 1)
            return accs

        zero = jnp.zeros((L,), jnp.float32)
        accs = lax.fori_loop(0, NCH // 2, chunk_pair, (zero,) * W)
        for w in range(W):
            accbuf[w, :] = accs[w]
        pltpu.sync_copy(accbuf, out_hbm.at[pl.ds(sp * W, W)])


def _tc_combine_body(parts_ref, xh_ref, out_ref):
    out_ref[...] = xh_ref[...] + jnp.sum(parts_ref[...], axis=1, keepdims=True)


_tc_combine = pl.pallas_call(
    _tc_combine_body,
    out_shape=jax.ShapeDtypeStruct((B, 1), jnp.float32),
)


def kernel(x, h, J, edge_idx_i, edge_idx_j):
    xq, xh = _tc_prep(x, h.reshape(1, N))
    parts = _sc_edges(xq.reshape(NSP, N * W), edge_idx_i, edge_idx_j, J)
    out = _tc_combine(parts, xh)
    return out.reshape(B)


# speedup vs baseline: 3.7645x; 3.7645x over previous
import functools

import jax
import jax.numpy as jnp
from jax import lax
from jax.experimental import pallas as pl
from jax.experimental.pallas import tpu as pltpu
from jax.experimental.pallas import tpu_sc as plsc

N = 10000
E = 160000
B = 1024

_info = plsc.get_sparse_core_info()
NC = _info.num_cores
NS = _info.num_subcores
L = _info.num_lanes
NW = NC * NS
EPW = E // NW
C = 40
NCHUNK = EPW // C

NB = 1024
NBLK = (N + NB - 1) // NB


EB = E // NBLK


def _tc_prep_body(x_ref, h_ref, j_ref, xt_ref, xh_ref, j16_ref):
    pid = pl.program_id(0)
    col = pid * NB + lax.broadcasted_iota(jnp.int32, (1, NB), 1)
    valid = col < N
    xb = jnp.where(valid, x_ref[...], 0.0)
    hb = jnp.where(valid, h_ref[...], 0.0)
    xt = xb.T
    xt_ref[...] = xt

    @pl.when(pid == 0)
    def _():
        xh_ref[...] = jnp.zeros_like(xh_ref)

    xh_ref[...] += lax.dot_general(
        hb, xt, (((1,), (0,)), ((), ())),
        preferred_element_type=jnp.float32)

    j16_ref[...] = jnp.broadcast_to(j_ref[...].reshape(EB, 1), (EB, 16))


_tc_prep = pl.pallas_call(
    _tc_prep_body,
    grid=(NBLK,),
    in_specs=[
        pl.BlockSpec((B, NB), lambda i: (0, i)),
        pl.BlockSpec((1, NB), lambda i: (0, i)),
        pl.BlockSpec((1, EB), lambda i: (0, i)),
    ],
    out_specs=[
        pl.BlockSpec((NB, B), lambda i: (i, 0)),
        pl.BlockSpec((1, B), lambda i: (0, 0)),
        pl.BlockSpec((EB, 16), lambda i: (i, 0)),
    ],
    out_shape=[
        jax.ShapeDtypeStruct((N, B), jnp.float32),
        jax.ShapeDtypeStruct((1, B), jnp.float32),
        jax.ShapeDtypeStruct((E, 16), jnp.float32),
    ],
)


R = 8
S = 5
AHEAD = S - 1
NSUB = EPW // R
NOUT = NSUB // S

QG = 16
NG = B // (QG * L)


@functools.partial(
    pl.kernel,
    mesh=plsc.VectorSubcoreMesh(core_axis_name="c", subcore_axis_name="s"),
    out_type=jax.ShapeDtypeStruct((NW, B), jnp.float32),
    scratch_types=[
        pltpu.VMEM((EPW,), jnp.int32),
        pltpu.VMEM((EPW,), jnp.int32),
        pltpu.VMEM((S, R, 16), jnp.float32),
        pltpu.VMEM((S, R, B), jnp.float32),
        pltpu.VMEM((S, R, B), jnp.float32),
        pltpu.VMEM((B,), jnp.float32),
        pltpu.SemaphoreType.DMA,
        pltpu.SemaphoreType.DMA,
        pltpu.SemaphoreType.DMA,
        pltpu.SemaphoreType.DMA,
        pltpu.SemaphoreType.DMA,
    ],
)
def _sc_edges(xt_hbm, j16_hbm, ei_hbm, ej_hbm, out_hbm,
              ei_v, ej_v, jv2, xi_v, xj_v, acc, *sems):
    wid = lax.axis_index("s") * NC + lax.axis_index("c")
    base = wid * EPW

    zero = jnp.zeros((L,), jnp.float32)
    for q in range(B // L):
        acc[pl.ds(q * L, L)] = zero

    cpi = pltpu.async_copy(ei_hbm.at[pl.ds(base, EPW)], ei_v, sems[0])
    cpj = pltpu.async_copy(ej_hbm.at[pl.ds(base, EPW)], ej_v, sems[1])
    cpi.wait()
    cpj.wait()

    def fire(sub, slot):
        pltpu.async_copy(xt_hbm.at[ei_v.at[pl.ds(sub * R, R)]],
                         xi_v.at[slot], sems[slot])
        pltpu.async_copy(xt_hbm.at[ej_v.at[pl.ds(sub * R, R)]],
                         xj_v.at[slot], sems[slot])
        pltpu.async_copy(j16_hbm.at[pl.ds(base + sub * R, R)],
                         jv2.at[slot], sems[slot])

    def drain(slot):
        pltpu.make_async_copy(xt_hbm.at[pl.ds(0, R)], xi_v.at[slot],
                              sems[slot]).wait()
        pltpu.make_async_copy(xt_hbm.at[pl.ds(0, R)], xj_v.at[slot],
                              sems[slot]).wait()
        pltpu.make_async_copy(j16_hbm.at[pl.ds(0, R)], jv2.at[slot],
                              sems[slot]).wait()

    for k in range(AHEAD):
        fire(k, k)

    def outer_body(o, carry):
        for k in range(S):
            s = o * S + k
            drain(k)
            @pl.when(s + AHEAD < NSUB)
            def _():
                fire(s + AHEAD, (k + AHEAD) % S)

            for g in range(NG):
                bq = g * QG * L

                def edge_body(e, accs, _bq=bq, _k=k):
                    jb = jv2[_k, e, :]
                    return tuple(
                        accs[q] + xi_v[_k, e, pl.ds(_bq + q * L, L)]
                        * xj_v[_k, e, pl.ds(_bq + q * L, L)] * jb
                        for q in range(QG))

                init = tuple(acc[pl.ds(bq + q * L, L)] for q in range(QG))
                accs = lax.fori_loop(0, R, edge_body, init)
                for q in range(QG):
                    acc[pl.ds(bq + q * L, L)] = accs[q]
        return carry

    lax.fori_loop(0, NOUT, outer_body, 0)
    pltpu.sync_copy(acc, out_hbm.at[wid])


def _tc_combine_body(parts_ref, xh_ref, out_ref):
    out_ref[...] = xh_ref[...] + jnp.sum(parts_ref[...], axis=0, keepdims=True)


_tc_combine = pl.pallas_call(
    _tc_combine_body,
    out_shape=jax.ShapeDtypeStruct((1, B), jnp.float32),
)


def kernel(x, h, J, edge_idx_i, edge_idx_j):
    xt, xh, j16 = _tc_prep(x, h.reshape(1, N), J.reshape(1, E))
    parts = _sc_edges(xt, j16, edge_idx_i, edge_idx_j)
    out = _tc_combine(parts, xh)
    return out.reshape(B)
